# Initial kernel scaffold; baseline (speedup 1.0000x reference)
import jax
import jax.numpy as jnp
from jax.experimental import pallas as pl


def kernel(x, gate_w):
    raise NotImplementedError("write your pallas kernel here")


# speedup vs baseline: 1.1196x; 1.1196x over previous
import jax
import jax.numpy as jnp
from jax.experimental import pallas as pl
from jax.experimental.pallas import tpu as pltpu

_B, _N, _D = 4, 4096, 4096
_E = 64
_K = 8
_T = 512


def _router_kernel(x_ref, w_ref, wts_ref, idx_ref, loss_ref, acc_ref):
    i = pl.program_id(0)
    nblocks = pl.num_programs(0)

    @pl.when(i == 0)
    def _init():
        acc_ref[...] = jnp.zeros_like(acc_ref)

    logits = jnp.dot(x_ref[...], w_ref[...], preferred_element_type=jnp.float32)
    m = jnp.max(logits, axis=-1, keepdims=True)
    p = jnp.exp(logits - m)
    s = jnp.sum(p, axis=-1, keepdims=True)
    probs = p / s

    acc_ref[...] += jnp.sum(probs, axis=0, keepdims=True)

    iota = jax.lax.broadcasted_iota(jnp.int32, probs.shape, 1)
    cur = probs
    vals, idxs = [], []
    for _ in range(_K):
        mk = jnp.max(cur, axis=-1, keepdims=True)
        ik = jnp.min(jnp.where(cur == mk, iota, _E), axis=-1, keepdims=True)
        vals.append(mk)
        idxs.append(ik)
        cur = jnp.where(iota == ik, -1.0, cur)
    v = jnp.concatenate(vals, axis=-1)
    wts_ref[...] = v / jnp.sum(v, axis=-1, keepdims=True)
    idx_ref[...] = jnp.concatenate(idxs, axis=-1)

    @pl.when(i == nblocks - 1)
    def _finish():
        usage = acc_ref[...] / (nblocks * _T)
        loss_ref[0, 0] = jnp.sum(usage * jnp.log(usage * _E + 1e-8))


def kernel(x, gate_w):
    tokens = _B * _N
    x2 = x.reshape(tokens, _D)
    w = gate_w.T
    grid = tokens // _T
    wts, idx, loss = pl.pallas_call(
        _router_kernel,
        grid=(grid,),
        in_specs=[
            pl.BlockSpec((_T, _D), lambda i: (i, 0)),
            pl.BlockSpec((_D, _E), lambda i: (0, 0)),
        ],
        out_specs=[
            pl.BlockSpec((_T, _K), lambda i: (i, 0)),
            pl.BlockSpec((_T, _K), lambda i: (i, 0)),
            pl.BlockSpec(memory_space=pltpu.SMEM),
        ],
        out_shape=[
            jax.ShapeDtypeStruct((tokens, _K), jnp.float32),
            jax.ShapeDtypeStruct((tokens, _K), jnp.int32),
            jax.ShapeDtypeStruct((1, 1), jnp.float32),
        ],
        scratch_shapes=[pltpu.VMEM((1, _E), jnp.float32)],
        compiler_params=pltpu.CompilerParams(
            dimension_semantics=("arbitrary",),
        ),
    )(x2, w)
    return (
        wts.reshape(_B, _N, _K),
        idx.reshape(_B, _N, _K),
        loss[0, 0],
    )


# speedup vs baseline: 1.2258x; 1.0948x over previous
import jax
import jax.numpy as jnp
from jax.experimental import pallas as pl
from jax.experimental.pallas import tpu as pltpu

_B, _N, _D = 4, 4096, 4096
_E = 64
_K = 8
_T = 512


def _router_kernel(x_ref, w_ref, wts_ref, idx_ref, loss_ref, acc_ref):
    i = pl.program_id(0)
    nblocks = pl.num_programs(0)

    @pl.when(i == 0)
    def _init():
        acc_ref[...] = jnp.zeros_like(acc_ref)

    logits = jnp.dot(x_ref[...], w_ref[...], preferred_element_type=jnp.float32)
    m = jnp.max(logits, axis=-1, keepdims=True)
    p = jnp.exp(logits - m)
    s = jnp.sum(p, axis=-1, keepdims=True)

    acc_ref[...] += jnp.sum(p * (1.0 / s), axis=0, keepdims=True)

    iota = jax.lax.broadcasted_iota(jnp.int32, p.shape, 1)
    packed = (jax.lax.bitcast_convert_type(p, jnp.int32) & ~(_E - 1)) | (
        (_E - 1) - iota
    )
    cur = packed
    vals, idxs = [], []
    for _ in range(_K):
        mk = jnp.max(cur, axis=-1, keepdims=True)
        idxs.append((_E - 1) - (mk & (_E - 1)))
        vals.append(jax.lax.bitcast_convert_type(mk & ~(_E - 1), jnp.float32))
        cur = jnp.where(cur == mk, jnp.int32(-(2**31)), cur)
    v = jnp.concatenate(vals, axis=-1)
    wts_ref[...] = v / jnp.sum(v, axis=-1, keepdims=True)
    idx_ref[...] = jnp.concatenate(idxs, axis=-1)

    @pl.when(i == nblocks - 1)
    def _finish():
        usage = acc_ref[...] / (nblocks * _T)
        loss_ref[0, 0] = jnp.sum(usage * jnp.log(usage * _E + 1e-8))


def kernel(x, gate_w):
    tokens = _B * _N
    x2 = x.reshape(tokens, _D)
    w = gate_w.T
    grid = tokens // _T
    wts, idx, loss = pl.pallas_call(
        _router_kernel,
        grid=(grid,),
        in_specs=[
            pl.BlockSpec((_T, _D), lambda i: (i, 0)),
            pl.BlockSpec((_D, _E), lambda i: (0, 0)),
        ],
        out_specs=[
            pl.BlockSpec((_T, _K), lambda i: (i, 0)),
            pl.BlockSpec((_T, _K), lambda i: (i, 0)),
            pl.BlockSpec(memory_space=pltpu.SMEM),
        ],
        out_shape=[
            jax.ShapeDtypeStruct((tokens, _K), jnp.float32),
            jax.ShapeDtypeStruct((tokens, _K), jnp.int32),
            jax.ShapeDtypeStruct((1, 1), jnp.float32),
        ],
        scratch_shapes=[pltpu.VMEM((1, _E), jnp.float32)],
        compiler_params=pltpu.CompilerParams(
            dimension_semantics=("arbitrary",),
        ),
    )(x2, w)
    return (
        wts.reshape(_B, _N, _K),
        idx.reshape(_B, _N, _K),
        loss[0, 0],
    )
